# baseline (speedup 1.0000x reference)
import functools

import jax
import jax.numpy as jnp
from jax import lax
from jax.experimental import pallas as pl
from jax.experimental.pallas import tpu as pltpu
from jax.experimental.pallas import tpu_sc as plsc

BATCH = 4096
SEQ = 200
HIDDEN = 64
NUM_WORKERS = 32
B_PER_W = BATCH // NUM_WORKERS
G = 2
NG = B_PER_W // G
C0, C1 = 128, SEQ - 128


@jax.jit
def _embed(tokens, table):
    mesh = plsc.VectorSubcoreMesh(core_axis_name="c", subcore_axis_name="s")

    @functools.partial(
        pl.kernel,
        mesh=mesh,
        compiler_params=pltpu.CompilerParams(use_tc_tiling_on_sc=False),
        out_type=jax.ShapeDtypeStruct((BATCH, SEQ, HIDDEN), jnp.float32),
        scratch_types=[
            pltpu.VMEM((B_PER_W * SEQ,), jnp.int32),
            pltpu.VMEM((G, SEQ, HIDDEN), jnp.float32),
            pltpu.VMEM((G, SEQ, HIDDEN), jnp.float32),
            pltpu.SemaphoreType.DMA,
            pltpu.SemaphoreType.DMA,
            pltpu.SemaphoreType.DMA,
            pltpu.SemaphoreType.DMA,
        ],
    )
    def k(tok_hbm, table_hbm, out_hbm, idx_v, buf_a, buf_b, gsem_a, gsem_b,
          osem_a, osem_b):
        wid = lax.axis_index("s") * 2 + lax.axis_index("c")
        b0 = wid * B_PER_W
        for r in range(B_PER_W):
            pltpu.async_copy(
                tok_hbm.at[b0 + r, :], idx_v.at[pl.ds(r * SEQ, SEQ)], gsem_a
            )
        for r in range(B_PER_W):
            pltpu.make_async_copy(
                tok_hbm.at[b0 + r, :], idx_v.at[pl.ds(r * SEQ, SEQ)], gsem_a
            ).wait()
        idx_flat = idx_v

        def fire(g, buf, gsem):
            for j in range(G):
                off = (g * G + j) * SEQ
                pltpu.async_copy(
                    table_hbm.at[idx_flat.at[pl.ds(off, C0)]],
                    buf.at[j, pl.ds(0, C0)],
                    gsem,
                )
                pltpu.async_copy(
                    table_hbm.at[idx_flat.at[pl.ds(off + C0, C1)]],
                    buf.at[j, pl.ds(C0, C1)],
                    gsem,
                )

        def drain(buf, gsem):
            for j in range(G):
                pltpu.make_async_copy(
                    table_hbm.at[idx_flat.at[pl.ds(0, C0)]],
                    buf.at[j, pl.ds(0, C0)],
                    gsem,
                ).wait()
                pltpu.make_async_copy(
                    table_hbm.at[idx_flat.at[pl.ds(0, C1)]],
                    buf.at[j, pl.ds(C0, C1)],
                    gsem,
                ).wait()

        def out_slice(g):
            return out_hbm.at[pl.ds(b0 + g * G, G)]

        def store(g, buf, osem):
            pltpu.async_copy(buf, out_slice(g), osem)

        def store_wait(g, buf, osem):
            pltpu.make_async_copy(buf, out_slice(g), osem).wait()

        fire(0, buf_a, gsem_a)
        fire(1, buf_b, gsem_b)
        drain(buf_a, gsem_a)
        store(0, buf_a, osem_a)

        def body(i, carry):
            store_wait(2 * i, buf_a, osem_a)
            fire(2 * i + 2, buf_a, gsem_a)
            drain(buf_b, gsem_b)
            store(2 * i + 1, buf_b, osem_b)
            store_wait(2 * i + 1, buf_b, osem_b)
            fire(2 * i + 3, buf_b, gsem_b)
            drain(buf_a, gsem_a)
            store(2 * i + 2, buf_a, osem_a)
            return carry

        lax.fori_loop(0, (NG - 2) // 2, body, 0)

        drain(buf_b, gsem_b)
        store(NG - 1, buf_b, osem_b)
        store_wait(NG - 2, buf_a, osem_a)
        store_wait(NG - 1, buf_b, osem_b)

    return k(tokens, table)


def kernel(tokens, embedding_weight):
    return _embed(tokens.astype(jnp.int32), embedding_weight)


# speedup vs baseline: 1.0021x; 1.0021x over previous
import functools

import jax
import jax.numpy as jnp
from jax import lax
from jax.experimental import pallas as pl
from jax.experimental.pallas import tpu as pltpu
from jax.experimental.pallas import tpu_sc as plsc

BATCH = 4096
SEQ = 200
HIDDEN = 64
NUM_WORKERS = 32
B_PER_W = BATCH // NUM_WORKERS
G = 2
NG = B_PER_W // G
C0, C1 = 128, SEQ - 128


@jax.jit
def _embed(tokens, table):
    mesh = plsc.VectorSubcoreMesh(core_axis_name="c", subcore_axis_name="s")

    @functools.partial(
        pl.kernel,
        mesh=mesh,
        compiler_params=pltpu.CompilerParams(use_tc_tiling_on_sc=False),
        out_type=jax.ShapeDtypeStruct((BATCH, SEQ, HIDDEN), jnp.float32),
        scratch_types=[
            pltpu.VMEM((B_PER_W * SEQ,), jnp.int32),
            pltpu.VMEM((G, SEQ, HIDDEN), jnp.float32),
            pltpu.VMEM((G, SEQ, HIDDEN), jnp.float32),
            pltpu.SemaphoreType.DMA,
            pltpu.SemaphoreType.DMA,
            pltpu.SemaphoreType.DMA,
            pltpu.SemaphoreType.DMA,
        ],
    )
    def k(tok_hbm, table_hbm, out_hbm, idx_v, buf_a, buf_b, gsem_a, gsem_b,
          osem_a, osem_b):
        wid = lax.axis_index("s") * 2 + lax.axis_index("c")
        b0 = wid * B_PER_W
        pltpu.sync_copy(tok_hbm.at[pl.ds(b0 * SEQ, B_PER_W * SEQ)], idx_v)
        idx_flat = idx_v

        def fire(g, buf, gsem):
            for j in range(G):
                off = (g * G + j) * SEQ
                pltpu.async_copy(
                    table_hbm.at[idx_flat.at[pl.ds(off, C0)]],
                    buf.at[j, pl.ds(0, C0)],
                    gsem,
                )
                pltpu.async_copy(
                    table_hbm.at[idx_flat.at[pl.ds(off + C0, C1)]],
                    buf.at[j, pl.ds(C0, C1)],
                    gsem,
                )

        def drain(buf, gsem):
            for j in range(G):
                pltpu.make_async_copy(
                    table_hbm.at[idx_flat.at[pl.ds(0, C0)]],
                    buf.at[j, pl.ds(0, C0)],
                    gsem,
                ).wait()
                pltpu.make_async_copy(
                    table_hbm.at[idx_flat.at[pl.ds(0, C1)]],
                    buf.at[j, pl.ds(C0, C1)],
                    gsem,
                ).wait()

        def out_slice(g):
            return out_hbm.at[pl.ds(b0 + g * G, G)]

        def store(g, buf, osem):
            pltpu.async_copy(buf, out_slice(g), osem)

        def store_wait(g, buf, osem):
            pltpu.make_async_copy(buf, out_slice(g), osem).wait()

        fire(0, buf_a, gsem_a)
        fire(1, buf_b, gsem_b)
        drain(buf_a, gsem_a)
        store(0, buf_a, osem_a)

        def body(i, carry):
            store_wait(2 * i, buf_a, osem_a)
            fire(2 * i + 2, buf_a, gsem_a)
            drain(buf_b, gsem_b)
            store(2 * i + 1, buf_b, osem_b)
            store_wait(2 * i + 1, buf_b, osem_b)
            fire(2 * i + 3, buf_b, gsem_b)
            drain(buf_a, gsem_a)
            store(2 * i + 2, buf_a, osem_a)
            return carry

        lax.fori_loop(0, (NG - 2) // 2, body, 0)

        drain(buf_b, gsem_b)
        store(NG - 1, buf_b, osem_b)
        store_wait(NG - 2, buf_a, osem_a)
        store_wait(NG - 1, buf_b, osem_b)

    return k(tokens, table)


def kernel(tokens, embedding_weight):
    return _embed(tokens.astype(jnp.int32).ravel(), embedding_weight)


# speedup vs baseline: 1.3332x; 1.3304x over previous
import functools

import jax
import jax.numpy as jnp
from jax import lax
from jax.experimental import pallas as pl
from jax.experimental.pallas import tpu as pltpu
from jax.experimental.pallas import tpu_sc as plsc

BATCH = 4096
SEQ = 200
HIDDEN = 64
PAD = 2 * HIDDEN
NUM_TOKENS = BATCH * SEQ
NUM_WORKERS = 32
B_PER_W = BATCH // NUM_WORKERS
G = 2
ROWS_G = G * SEQ
NG = B_PER_W // G
C0, C1 = 128, SEQ - 128


@jax.jit
def _embed(tokens, table):
    mesh = plsc.VectorSubcoreMesh(core_axis_name="c", subcore_axis_name="s")

    @functools.partial(
        pl.kernel,
        mesh=mesh,
        compiler_params=pltpu.CompilerParams(use_tc_tiling_on_sc=False),
        out_type=jax.ShapeDtypeStruct((NUM_TOKENS, PAD), jnp.float32),
        scratch_types=[
            pltpu.VMEM((B_PER_W * SEQ,), jnp.int32),
            pltpu.VMEM((ROWS_G, HIDDEN), jnp.float32),
            pltpu.VMEM((ROWS_G, HIDDEN), jnp.float32),
            pltpu.SemaphoreType.DMA,
            pltpu.SemaphoreType.DMA,
            pltpu.SemaphoreType.DMA,
            pltpu.SemaphoreType.DMA,
        ],
    )
    def k(tok_hbm, table_hbm, out_hbm, idx_v, buf_a, buf_b, gsem_a, gsem_b,
          osem_a, osem_b):
        wid = lax.axis_index("s") * 2 + lax.axis_index("c")
        t0 = wid * B_PER_W * SEQ
        pltpu.sync_copy(tok_hbm.at[pl.ds(t0, B_PER_W * SEQ)], idx_v)

        def fire(g, buf, gsem):
            for j in range(G):
                off = (g * G + j) * SEQ
                pltpu.async_copy(
                    table_hbm.at[idx_v.at[pl.ds(off, C0)]],
                    buf.at[pl.ds(j * SEQ, C0)],
                    gsem,
                )
                pltpu.async_copy(
                    table_hbm.at[idx_v.at[pl.ds(off + C0, C1)]],
                    buf.at[pl.ds(j * SEQ + C0, C1)],
                    gsem,
                )

        def drain(buf, gsem):
            for j in range(G):
                pltpu.make_async_copy(
                    table_hbm.at[idx_v.at[pl.ds(0, C0)]],
                    buf.at[pl.ds(j * SEQ, C0)],
                    gsem,
                ).wait()
                pltpu.make_async_copy(
                    table_hbm.at[idx_v.at[pl.ds(0, C1)]],
                    buf.at[pl.ds(j * SEQ + C0, C1)],
                    gsem,
                ).wait()

        def out_slice(g):
            return out_hbm.at[pl.ds(t0 + g * ROWS_G, ROWS_G), pl.ds(0, HIDDEN)]

        def store(g, buf, osem):
            pltpu.async_copy(buf, out_slice(g), osem)

        def store_wait(g, buf, osem):
            pltpu.make_async_copy(buf, out_slice(g), osem).wait()

        fire(0, buf_a, gsem_a)
        fire(1, buf_b, gsem_b)
        drain(buf_a, gsem_a)
        store(0, buf_a, osem_a)

        def body(i, carry):
            store_wait(2 * i, buf_a, osem_a)
            fire(2 * i + 2, buf_a, gsem_a)
            drain(buf_b, gsem_b)
            store(2 * i + 1, buf_b, osem_b)
            store_wait(2 * i + 1, buf_b, osem_b)
            fire(2 * i + 3, buf_b, gsem_b)
            drain(buf_a, gsem_a)
            store(2 * i + 2, buf_a, osem_a)
            return carry

        lax.fori_loop(0, (NG - 2) // 2, body, 0)

        drain(buf_b, gsem_b)
        store(NG - 1, buf_b, osem_b)
        store_wait(NG - 2, buf_a, osem_a)
        store_wait(NG - 1, buf_b, osem_b)

    return k(tokens, table)


def kernel(tokens, embedding_weight):
    out = _embed(tokens.astype(jnp.int32).ravel(), embedding_weight)
    return out.reshape(BATCH, SEQ, PAD)[..., :HIDDEN]
